# Initial kernel scaffold; baseline (speedup 1.0000x reference)
import jax
import jax.numpy as jnp
from jax.experimental import pallas as pl


def kernel(x, W, A, Bm):
    raise NotImplementedError("write your pallas kernel here")


# speedup vs baseline: 5.9965x; 5.9965x over previous
import functools

import jax
import jax.numpy as jnp
from jax import lax
from jax.experimental import pallas as pl
from jax.experimental.pallas import tpu as pltpu
from jax.experimental.pallas import tpu_sc as plsc

_LANES = 16
_CHUNK = 128


def _sc_lora_embed(n_tokens, dim, rank, scaling):
    info = plsc.get_sparse_core_info()
    nc, ns = info.num_cores, info.num_subcores
    nw = nc * ns
    assert n_tokens % (nw * _CHUNK) == 0
    tpw = n_tokens // nw
    n_chunks = tpw // _CHUNK
    d_vecs = dim // _LANES

    mesh = plsc.VectorSubcoreMesh(core_axis_name="c", subcore_axis_name="s")

    @functools.partial(
        pl.kernel,
        out_type=jax.ShapeDtypeStruct((n_tokens, dim), jnp.float32),
        mesh=mesh,
        scratch_types=[
            pltpu.VMEM((_CHUNK,), jnp.int32),
            pltpu.VMEM((_CHUNK, rank), jnp.float32),
            pltpu.VMEM((_CHUNK, dim), jnp.float32),
            pltpu.VMEM((rank, dim), jnp.float32),
            pltpu.SemaphoreType.DMA,
            pltpu.SemaphoreType.DMA,
        ],
        compiler_params=pltpu.CompilerParams(use_tc_tiling_on_sc=False),
    )
    def k(x_hbm, w_hbm, a_hbm, bm_hbm, out_hbm, idx_v, a_v, w_v, bm_v,
          sem_a, sem_w):
        wid = lax.axis_index("s") * nc + lax.axis_index("c")
        base = wid * tpw
        pltpu.sync_copy(bm_hbm, bm_v)

        def chunk_body(ci, _):
            tok0 = base + ci * _CHUNK
            pltpu.sync_copy(x_hbm.at[pl.ds(tok0, _CHUNK)], idx_v)
            cp_a = pltpu.async_copy(a_hbm.at[idx_v], a_v, sem_a)
            cp_w = pltpu.async_copy(w_hbm.at[idx_v], w_v, sem_w)
            cp_a.wait()
            cp_w.wait()

            for half in range(d_vecs // 2):
                d0 = half * 2 * _LANES
                bm_rows = [
                    (bm_v[r, pl.ds(d0, _LANES)],
                     bm_v[r, pl.ds(d0 + _LANES, _LANES)])
                    for r in range(rank)
                ]

                def tok_body(t, _, d0=d0, bm_rows=bm_rows):
                    a_vec = a_v[t, pl.ds(0, rank)] * scaling
                    acc0 = w_v[t, pl.ds(d0, _LANES)]
                    acc1 = w_v[t, pl.ds(d0 + _LANES, _LANES)]
                    for r in range(rank):
                        s = a_vec[r]
                        acc0 = acc0 + bm_rows[r][0] * s
                        acc1 = acc1 + bm_rows[r][1] * s
                    w_v[t, pl.ds(d0, _LANES)] = acc0
                    w_v[t, pl.ds(d0 + _LANES, _LANES)] = acc1
                    return 0

                lax.fori_loop(0, _CHUNK, tok_body, 0)

            pltpu.sync_copy(w_v, out_hbm.at[pl.ds(tok0, _CHUNK)])
            return 0

        lax.fori_loop(0, n_chunks, chunk_body, 0)

    return k


def kernel(x, W, A, Bm):
    batch, seq = x.shape
    vocab, dim = W.shape
    rank = A.shape[1]
    scaling = 16.0 / rank
    xf = x.reshape(-1).astype(jnp.int32)
    k = _sc_lora_embed(xf.shape[0], dim, rank, scaling)
    out = k(xf, W, A, Bm)
    return out.reshape(batch, seq, dim)


# speedup vs baseline: 6.5992x; 1.1005x over previous
import functools

import jax
import jax.numpy as jnp
from jax import lax
from jax.experimental import pallas as pl
from jax.experimental.pallas import tpu as pltpu
from jax.experimental.pallas import tpu_sc as plsc

_LANES = 16
_CHUNK = 128


def _sc_lora_embed(n_tokens, dim, rank, scaling):
    info = plsc.get_sparse_core_info()
    nc, ns = info.num_cores, info.num_subcores
    nw = nc * ns
    assert n_tokens % (nw * 2 * _CHUNK) == 0
    tpw = n_tokens // nw
    n_pairs = tpw // (2 * _CHUNK)
    d_vecs = dim // _LANES

    mesh = plsc.VectorSubcoreMesh(core_axis_name="c", subcore_axis_name="s")

    @functools.partial(
        pl.kernel,
        out_type=jax.ShapeDtypeStruct((n_tokens, dim), jnp.float32),
        mesh=mesh,
        scratch_types=[
            pltpu.VMEM((tpw,), jnp.int32),
            pltpu.VMEM((_CHUNK, rank), jnp.float32),
            pltpu.VMEM((_CHUNK, rank), jnp.float32),
            pltpu.VMEM((_CHUNK, dim), jnp.float32),
            pltpu.VMEM((_CHUNK, dim), jnp.float32),
            pltpu.VMEM((rank, dim), jnp.float32),
            pltpu.SemaphoreType.DMA,
            pltpu.SemaphoreType.DMA,
            pltpu.SemaphoreType.DMA,
            pltpu.SemaphoreType.DMA,
            pltpu.SemaphoreType.DMA,
            pltpu.SemaphoreType.DMA,
        ],
        compiler_params=pltpu.CompilerParams(use_tc_tiling_on_sc=False),
    )
    def k(x_hbm, w_hbm, a_hbm, bm_hbm, out_hbm, idx_v, a_v0, a_v1, w_v0,
          w_v1, bm_v, sem_a0, sem_a1, sem_w0, sem_w1, sem_o0, sem_o1):
        wid = lax.axis_index("s") * nc + lax.axis_index("c")
        base = wid * tpw
        a_bufs = (a_v0, a_v1)
        w_bufs = (w_v0, w_v1)
        sems_a = (sem_a0, sem_a1)
        sems_w = (sem_w0, sem_w1)
        sems_o = (sem_o0, sem_o1)

        pltpu.sync_copy(bm_hbm, bm_v)
        pltpu.sync_copy(x_hbm.at[pl.ds(base, tpw)], idx_v)

        def gather(chunk, buf):
            idx = idx_v.at[pl.ds(chunk * _CHUNK, _CHUNK)]
            pltpu.async_copy(a_hbm.at[idx], a_bufs[buf], sems_a[buf])
            pltpu.async_copy(w_hbm.at[idx], w_bufs[buf], sems_w[buf])

        def gather_wait(buf):
            pltpu.make_async_copy(a_hbm.at[idx_v.at[pl.ds(0, _CHUNK)]],
                                  a_bufs[buf], sems_a[buf]).wait()
            pltpu.make_async_copy(w_hbm.at[idx_v.at[pl.ds(0, _CHUNK)]],
                                  w_bufs[buf], sems_w[buf]).wait()

        def scatter(chunk, buf):
            pltpu.async_copy(
                w_bufs[buf],
                out_hbm.at[pl.ds(base + chunk * _CHUNK, _CHUNK)],
                sems_o[buf])

        def scatter_wait(buf):
            pltpu.make_async_copy(
                w_bufs[buf], out_hbm.at[pl.ds(base, _CHUNK)],
                sems_o[buf]).wait()

        def compute(buf):
            a_b, w_b = a_bufs[buf], w_bufs[buf]
            for half in range(d_vecs // 2):
                d0 = half * 2 * _LANES
                bm_rows = [
                    (bm_v[r, pl.ds(d0, _LANES)],
                     bm_v[r, pl.ds(d0 + _LANES, _LANES)])
                    for r in range(rank)
                ]

                def tok_body(t, _, d0=d0, bm_rows=bm_rows):
                    a_vec = a_b[t, pl.ds(0, rank)]
                    if scaling != 1.0:
                        a_vec = a_vec * scaling
                    acc0 = w_b[t, pl.ds(d0, _LANES)]
                    acc1 = w_b[t, pl.ds(d0 + _LANES, _LANES)]
                    for r in range(rank):
                        s = a_vec[r]
                        acc0 = acc0 + bm_rows[r][0] * s
                        acc1 = acc1 + bm_rows[r][1] * s
                    w_b[t, pl.ds(d0, _LANES)] = acc0
                    w_b[t, pl.ds(d0 + _LANES, _LANES)] = acc1
                    return 0

                lax.fori_loop(0, _CHUNK, tok_body, 0, unroll=2)

        gather(0, 0)

        def pair_body(ci, _):
            even = 2 * ci
            @pl.when(ci > 0)
            def _():
                scatter_wait(1)
            gather(even + 1, 1)
            gather_wait(0)
            compute(0)
            scatter(even, 0)
            gather_wait(1)
            scatter_wait(0)

            @pl.when(ci < n_pairs - 1)
            def _():
                gather(even + 2, 0)
            compute(1)
            scatter(even + 1, 1)
            return 0

        lax.fori_loop(0, n_pairs, pair_body, 0)
        scatter_wait(1)

    return k


def kernel(x, W, A, Bm):
    batch, seq = x.shape
    vocab, dim = W.shape
    rank = A.shape[1]
    scaling = 16.0 / rank
    xf = x.reshape(-1).astype(jnp.int32)
    k = _sc_lora_embed(xf.shape[0], dim, rank, scaling)
    out = k(xf, W, A, Bm)
    return out.reshape(batch, seq, dim)
